# baseline (speedup 1.0000x reference)
import math

import jax
import jax.numpy as jnp
from jax import lax
from jax.experimental import pallas as pl
from jax.experimental.pallas import tpu as pltpu
from jax.experimental.pallas import tpu_sc as plsc

VOCAB = 1000000
EMBED = 64
WINDOW = 200
BATCH = 4096

NC, NS, LANES = 2, 16, 16
NW = NC * NS
SEQ_PER_W = BATCH // NW
ROWS = WINDOW
VECS_PER_ROW = EMBED // LANES
SCALE = math.sqrt(EMBED)


def _body(x_hbm, table_hbm, pe_hbm, out_hbm,
          idx_v, pe_v, rows0, rows1, out0, out1,
          gsem0, gsem1, wsem0, wsem1):
    wid = lax.axis_index("s") * NC + lax.axis_index("c")
    seq0 = wid * SEQ_PER_W

    pltpu.sync_copy(pe_hbm, pe_v)
    pltpu.sync_copy(x_hbm.at[pl.ds(seq0 * WINDOW, SEQ_PER_W * WINDOW)], idx_v)

    bufs = ((rows0, out0, gsem0, wsem0), (rows1, out1, gsem1, wsem1))

    def start_gather(i, rows_b, gsem_b):
        pltpu.async_copy(table_hbm.at[idx_v.at[pl.ds(i * ROWS, ROWS)]],
                         rows_b, gsem_b)

    for b, (rows_b, out_b, gsem_b, wsem_b) in enumerate(bufs):
        start_gather(b, rows_b, gsem_b)
        pltpu.async_copy(out_b, out_hbm.at[pl.ds((seq0 + b) * ROWS, ROWS)],
                         wsem_b)

    def step(j, _):
        for b, (rows_b, out_b, gsem_b, wsem_b) in enumerate(bufs):
            i = 2 * j + b
            pltpu.make_async_copy(
                table_hbm.at[idx_v.at[pl.ds(i * ROWS, ROWS)]],
                rows_b, gsem_b).wait()
            pltpu.make_async_copy(
                out_b, out_hbm.at[pl.ds((seq0 + i) * ROWS, ROWS)],
                wsem_b).wait()

            def row_step(r, _):
                for k in range(VECS_PER_ROW):
                    sl = pl.ds(k * LANES, LANES)
                    out_b[r, sl] = rows_b[r, sl] * SCALE + pe_v[r, sl]
                return ()

            lax.fori_loop(0, ROWS, row_step, (), unroll=4)

            @pl.when(i + 2 < SEQ_PER_W)
            def _():
                start_gather(i + 2, rows_b, gsem_b)

            pltpu.async_copy(out_b,
                             out_hbm.at[pl.ds((seq0 + i) * ROWS, ROWS)],
                             wsem_b)
        return ()

    lax.fori_loop(0, SEQ_PER_W // 2, step, (), unroll=False)

    for b, (rows_b, out_b, gsem_b, wsem_b) in enumerate(bufs):
        i = SEQ_PER_W - 2 + b
        pltpu.make_async_copy(
            out_b, out_hbm.at[pl.ds((seq0 + i) * ROWS, ROWS)],
            wsem_b).wait()


def kernel(x, table, pos_encoding):
    xf = x.reshape(BATCH * WINDOW)
    pe = pos_encoding[:WINDOW, :]

    mesh = plsc.VectorSubcoreMesh(
        core_axis_name="c", subcore_axis_name="s",
        num_cores=NC, num_subcores=NS)

    out = pl.kernel(
        _body,
        out_type=jax.ShapeDtypeStruct((BATCH * WINDOW, EMBED), jnp.float32),
        mesh=mesh,
        scratch_types=[
            pltpu.VMEM((SEQ_PER_W * WINDOW,), jnp.int32),
            pltpu.VMEM((WINDOW, EMBED), jnp.float32),
            pltpu.VMEM((ROWS, EMBED), jnp.float32),
            pltpu.VMEM((ROWS, EMBED), jnp.float32),
            pltpu.VMEM((ROWS, EMBED), jnp.float32),
            pltpu.VMEM((ROWS, EMBED), jnp.float32),
            pltpu.SemaphoreType.DMA,
            pltpu.SemaphoreType.DMA,
            pltpu.SemaphoreType.DMA,
            pltpu.SemaphoreType.DMA,
        ],
        compiler_params=pltpu.CompilerParams(use_tc_tiling_on_sc=False),
    )(xf, table, pe)
    return out.reshape(BATCH, WINDOW, EMBED)


# speedup vs baseline: 1.1854x; 1.1854x over previous
import math

import jax
import jax.numpy as jnp
from jax import lax
from jax.experimental import pallas as pl
from jax.experimental.pallas import tpu as pltpu
from jax.experimental.pallas import tpu_sc as plsc

VOCAB = 1000000
EMBED = 64
EPAD = 128
WINDOW = 200
BATCH = 4096

NC, NS, LANES = 2, 16, 16
NW = NC * NS
SEQ_PER_W = BATCH // NW
PHASES = 2
SEQ_PER_PHASE = SEQ_PER_W // PHASES
ROWS = WINDOW
VECS_PER_ROW = EMBED // LANES
SCALE = math.sqrt(EMBED)


def _body(x_hbm, table_hbm, pe_hbm, out_hbm,
          idx_v, pe_v, rows0, rows1, out0, out1,
          gsem0, gsem1, wsem0, wsem1):
    wid = lax.axis_index("s") * NC + lax.axis_index("c")
    seq0 = wid * SEQ_PER_W

    pltpu.sync_copy(pe_hbm, pe_v)

    bufs = ((rows0, out0, gsem0, wsem0), (rows1, out1, gsem1, wsem1))

    def start_gather(c, rows_b, gsem_b):
        pltpu.async_copy(table_hbm.at[idx_v.at[pl.ds(c * ROWS, ROWS)]],
                         rows_b, gsem_b)

    for b, (rows_b, out_b, gsem_b, wsem_b) in enumerate(bufs):
        pltpu.async_copy(out_b, out_hbm.at[seq0 + b], wsem_b)

    for phase in range(PHASES):
        pbase = seq0 + phase * SEQ_PER_PHASE
        pltpu.sync_copy(
            x_hbm.at[pl.ds(pbase * WINDOW, SEQ_PER_PHASE * WINDOW)], idx_v)
        for b, (rows_b, out_b, gsem_b, wsem_b) in enumerate(bufs):
            start_gather(b, rows_b, gsem_b)

        def step(j, _):
            for b, (rows_b, out_b, gsem_b, wsem_b) in enumerate(bufs):
                c = 2 * j + b
                pltpu.make_async_copy(
                    table_hbm.at[idx_v.at[pl.ds(c * ROWS, ROWS)]],
                    rows_b, gsem_b).wait()
                pltpu.make_async_copy(
                    out_b, out_hbm.at[pbase + c], wsem_b).wait()

                def row_step(r, _):
                    for k in range(VECS_PER_ROW):
                        sl = pl.ds(k * LANES, LANES)
                        out_b[r, sl] = (rows_b[r, sl] * SCALE
                                        + pe_v[pl.ds(r * EMBED + k * LANES,
                                                     LANES)])
                    return ()

                lax.fori_loop(0, ROWS, row_step, (), unroll=4)

                @pl.when(c + 2 < SEQ_PER_PHASE)
                def _():
                    start_gather(c + 2, rows_b, gsem_b)

                pltpu.async_copy(out_b, out_hbm.at[pbase + c], wsem_b)
            return ()

        lax.fori_loop(0, SEQ_PER_PHASE // 2, step, (), unroll=False)

    for b, (rows_b, out_b, gsem_b, wsem_b) in enumerate(bufs):
        c = SEQ_PER_PHASE - 2 + b
        pltpu.make_async_copy(
            out_b, out_hbm.at[seq0 + SEQ_PER_PHASE + c], wsem_b).wait()


def kernel(x, table, pos_encoding):
    xf = x.reshape(BATCH * WINDOW)
    pe = pos_encoding[:WINDOW, :].reshape(WINDOW * EMBED)
    tpad = jnp.pad(table, ((0, 0), (0, EPAD - EMBED)))

    mesh = plsc.VectorSubcoreMesh(
        core_axis_name="c", subcore_axis_name="s",
        num_cores=NC, num_subcores=NS)

    out = pl.kernel(
        _body,
        out_type=jax.ShapeDtypeStruct((BATCH, WINDOW, EMBED), jnp.float32),
        mesh=mesh,
        scratch_types=[
            pltpu.VMEM((SEQ_PER_PHASE * WINDOW,), jnp.int32),
            pltpu.VMEM((WINDOW * EMBED,), jnp.float32),
            pltpu.VMEM((ROWS, EPAD), jnp.float32),
            pltpu.VMEM((ROWS, EPAD), jnp.float32),
            pltpu.VMEM((ROWS, EMBED), jnp.float32),
            pltpu.VMEM((ROWS, EMBED), jnp.float32),
            pltpu.SemaphoreType.DMA,
            pltpu.SemaphoreType.DMA,
            pltpu.SemaphoreType.DMA,
            pltpu.SemaphoreType.DMA,
        ],
    )(xf, tpad, pe)
    return out
